# baseline (speedup 1.0000x reference)
import functools

import jax
import jax.numpy as jnp
from jax import lax
from jax.experimental import pallas as pl
from jax.experimental.pallas import tpu as pltpu
from jax.experimental.pallas import tpu_sc as plsc

D_EMBED = 128
TABLE = 32
N = 262144

NC = 2
NS = 16
NW = NC * NS
PTS = N // NW
B = 64
NBATCH = PTS // B
LANES = 16
CH = D_EMBED // LANES
WORDS = D_EMBED


def _pair_terms(v, stride):
    iv = v * (TABLE / 2.0) + (TABLE - 1.0) / 2.0
    i0 = (iv + 1.0).astype(jnp.int32) - 1
    f0 = i0.astype(jnp.float32)
    w1 = iv - f0
    w0 = 1.0 - w1
    bb = jnp.clip(i0, 0, TABLE - 2)
    zero = jnp.zeros_like(w0)
    wlow = jnp.where(i0 == bb, w0, jnp.where(i0 < 0, w1, zero))
    whigh = jnp.where(i0 == bb, w1, jnp.where(i0 > TABLE - 2, w0, zero))
    return bb * stride, wlow, whigh


def _axis_terms(v, stride):
    iv = v * (TABLE / 2.0) + (TABLE - 1.0) / 2.0
    i0 = (iv + 1.0).astype(jnp.int32) - 1
    f0 = i0.astype(jnp.float32)
    w1 = iv - f0
    w0 = 1.0 - w1
    w0 = jnp.where(i0 >= 0, w0, 0.0)
    w1 = jnp.where(i0 <= TABLE - 2, w1, 0.0)
    e0 = jnp.maximum(i0, 0) * stride
    e1 = jnp.minimum(i0 + 1, TABLE - 1) * stride
    return e0, e1, w0, w1


def _body(xs_hbm, ys_hbm, zs_hbm, tab_hbm, out_hbm,
          xs_v, ys_v, zs_v, idx_v, w_v, rows_v, out_v,
          gsem0, gsem1, osem0, osem1):
    c = lax.axis_index("c")
    s = lax.axis_index("s")
    wid = s * NC + c
    base = wid * PTS
    gsem = (gsem0, gsem1)
    osem = (osem0, osem1)

    pltpu.sync_copy(xs_hbm.at[pl.ds(base, PTS)], xs_v)
    pltpu.sync_copy(ys_hbm.at[pl.ds(base, PTS)], ys_v)
    pltpu.sync_copy(zs_hbm.at[pl.ds(base, PTS)], zs_v)

    def compute_and_fire(i, p):
        b0 = i * B
        for sl in range(B // LANES):
            src = pl.ds(b0 + sl * LANES, LANES)
            dst = pl.ds(sl * LANES, LANES)
            xv = jnp.clip(xs_v[src], -1.0, 1.0)
            yv = jnp.clip(ys_v[src], -1.0, 1.0)
            zv = jnp.clip(zs_v[src], -1.0, 1.0)
            ex, wx0, wx1 = _pair_terms(xv, 1)
            ey0, ey1, wy0, wy1 = _axis_terms(yv, TABLE)
            ez0, ez1, wz0, wz1 = _axis_terms(zv, TABLE * TABLE)
            for k in range(4):
                ey, wy = (ey1, wy1) if (k & 1) else (ey0, wy0)
                ez, wz = (ez1, wz1) if (k & 2) else (ez0, wz0)
                wyz = wy * wz
                idx_v[p, k, dst] = ez + ey + ex
                w_v[p, 2 * k, dst] = wyz * wx0
                w_v[p, 2 * k + 1, dst] = wyz * wx1
        for k in range(4):
            pltpu.async_copy(tab_hbm.at[idx_v.at[p, k]], rows_v.at[p, k],
                             gsem[p])

    def wait_gathers(p):
        for k in range(4):
            pltpu.make_async_copy(tab_hbm.at[idx_v.at[p, k]],
                                  rows_v.at[p, k], gsem[p]).wait()

    def wait_out(i, p):
        pltpu.make_async_copy(out_v.at[p], out_hbm.at[pl.ds(base + i * B, B)],
                              osem[p]).wait()

    def blend(i, p):
        def do_point(b, lane):
            blk = lax.mul(lax.div(b, LANES), LANES)
            accs = [jnp.zeros((LANES,), jnp.float32)] * CH
            for k in range(4):
                wrow_lo = w_v[p, 2 * k, pl.ds(blk, LANES)]
                wrow_hi = w_v[p, 2 * k + 1, pl.ds(blk, LANES)]
                wlo = wrow_lo.at[lane].get(mode="promise_in_bounds")
                whi = wrow_hi.at[lane].get(mode="promise_in_bounds")
                for t in range(CH):
                    packed = rows_v[p, k, b, pl.ds(t * LANES, LANES)]
                    a = lax.bitcast_convert_type(packed << 16, jnp.float32)
                    bv = lax.bitcast_convert_type(
                        packed & jnp.int32(-65536), jnp.float32)
                    accs[t] = accs[t] + (wlo * a + whi * bv)
            for t in range(CH):
                out_v[p, b, pl.ds(t * LANES, LANES)] = accs[t]

        def pt_body(h, _):
            b = 8 * h
            lane = jnp.full((LANES,), lax.rem(b, LANES), jnp.int32)
            for j in range(8):
                do_point(b + j, lane + j)
            return 0

        lax.fori_loop(0, B // 8, pt_body, 0)

    def half(i, p):
        @pl.when(i + 1 < NBATCH)
        def _():
            compute_and_fire(i + 1, 1 - p)

        wait_gathers(p)

        @pl.when(i >= 2)
        def _():
            wait_out(i, p)

        blend(i, p)
        pltpu.async_copy(out_v.at[p], out_hbm.at[pl.ds(base + i * B, B)],
                         osem[p])

    def round_body(g, carry):
        half(2 * g, 0)
        half(2 * g + 1, 1)
        return carry

    compute_and_fire(0, 0)
    lax.fori_loop(0, NBATCH // 2, round_body, 0)
    wait_out(NBATCH - 2, 0)
    wait_out(NBATCH - 1, 1)


@functools.partial(jax.jit, donate_argnums=())
def kernel(x, table):
    xc = jnp.clip(x, -1.0, 1.0)
    xs = xc[:, 0]
    ys = xc[:, 1]
    zs = xc[:, 2]
    S = TABLE * TABLE * TABLE
    tt = table.reshape(D_EMBED, S).T.astype(jnp.bfloat16)
    hi = jnp.concatenate([tt[1:], tt[-1:]], axis=0)
    pair = jnp.stack([tt, hi], axis=-1)
    tab_i32 = lax.bitcast_convert_type(pair, jnp.int32)

    run = functools.partial(
        pl.kernel,
        out_type=jax.ShapeDtypeStruct((N, D_EMBED), jnp.float32),
        mesh=plsc.VectorSubcoreMesh(core_axis_name="c", subcore_axis_name="s"),
        scratch_types=[
            pltpu.VMEM((PTS,), jnp.float32),
            pltpu.VMEM((PTS,), jnp.float32),
            pltpu.VMEM((PTS,), jnp.float32),
            pltpu.VMEM((2, 4, B), jnp.int32),
            pltpu.VMEM((2, 8, B), jnp.float32),
            pltpu.VMEM((2, 4, B, WORDS), jnp.int32),
            pltpu.VMEM((2, B, D_EMBED), jnp.float32),
            pltpu.SemaphoreType.DMA,
            pltpu.SemaphoreType.DMA,
            pltpu.SemaphoreType.DMA,
            pltpu.SemaphoreType.DMA,
        ],
    )(_body)
    return run(xs, ys, zs, tab_i32)


# speedup vs baseline: 1.0653x; 1.0653x over previous
import functools

import jax
import jax.numpy as jnp
from jax import lax
from jax.experimental import pallas as pl
from jax.experimental.pallas import tpu as pltpu
from jax.experimental.pallas import tpu_sc as plsc

D_EMBED = 128
TABLE = 32
N = 262144

NC = 2
NS = 16
NW = NC * NS
PTS = N // NW
B = 64
NBATCH = PTS // B
LANES = 16
CH = D_EMBED // LANES
WORDS = D_EMBED


def _pair_terms(v, stride):
    iv = v * (TABLE / 2.0) + (TABLE - 1.0) / 2.0
    i0 = (iv + 1.0).astype(jnp.int32) - 1
    f0 = i0.astype(jnp.float32)
    w1 = iv - f0
    w0 = 1.0 - w1
    bb = jnp.clip(i0, 0, TABLE - 2)
    zero = jnp.zeros_like(w0)
    wlow = jnp.where(i0 == bb, w0, jnp.where(i0 < 0, w1, zero))
    whigh = jnp.where(i0 == bb, w1, jnp.where(i0 > TABLE - 2, w0, zero))
    return bb * stride, wlow, whigh


def _axis_terms(v, stride):
    iv = v * (TABLE / 2.0) + (TABLE - 1.0) / 2.0
    i0 = (iv + 1.0).astype(jnp.int32) - 1
    f0 = i0.astype(jnp.float32)
    w1 = iv - f0
    w0 = 1.0 - w1
    w0 = jnp.where(i0 >= 0, w0, 0.0)
    w1 = jnp.where(i0 <= TABLE - 2, w1, 0.0)
    e0 = jnp.maximum(i0, 0) * stride
    e1 = jnp.minimum(i0 + 1, TABLE - 1) * stride
    return e0, e1, w0, w1


def _body(xs_hbm, ys_hbm, zs_hbm, tab_hbm, out_hbm,
          xs_v, ys_v, zs_v, idx_v, w_v, rows_v, out_v,
          gsem0, gsem1, osem0, osem1):
    c = lax.axis_index("c")
    s = lax.axis_index("s")
    wid = s * NC + c
    base = wid * PTS
    gsem = (gsem0, gsem1)
    osem = (osem0, osem1)

    pltpu.sync_copy(xs_hbm.at[pl.ds(base, PTS)], xs_v)
    pltpu.sync_copy(ys_hbm.at[pl.ds(base, PTS)], ys_v)
    pltpu.sync_copy(zs_hbm.at[pl.ds(base, PTS)], zs_v)

    def compute_and_fire(i, p):
        b0 = i * B
        for sl in range(B // LANES):
            src = pl.ds(b0 + sl * LANES, LANES)
            dst = pl.ds(sl * LANES, LANES)
            xv = jnp.clip(xs_v[src], -1.0, 1.0)
            yv = jnp.clip(ys_v[src], -1.0, 1.0)
            zv = jnp.clip(zs_v[src], -1.0, 1.0)
            ex, wx0, wx1 = _pair_terms(xv, 1)
            ey0, ey1, wy0, wy1 = _axis_terms(yv, TABLE)
            ez0, ez1, wz0, wz1 = _axis_terms(zv, TABLE * TABLE)
            for k in range(4):
                ey, wy = (ey1, wy1) if (k & 1) else (ey0, wy0)
                ez, wz = (ez1, wz1) if (k & 2) else (ez0, wz0)
                wyz = wy * wz
                idx_v[p, k, dst] = ez + ey + ex
                w_v[p, 2 * k, dst] = wyz * wx0
                w_v[p, 2 * k + 1, dst] = wyz * wx1
        for k in range(4):
            pltpu.async_copy(tab_hbm.at[idx_v.at[p, k]], rows_v.at[p, k],
                             gsem[p])

    def wait_gathers(p):
        for k in range(4):
            pltpu.make_async_copy(tab_hbm.at[idx_v.at[p, k]],
                                  rows_v.at[p, k], gsem[p]).wait()

    def wait_out(i, p):
        pltpu.make_async_copy(out_v.at[p], out_hbm.at[pl.ds(base + i * B, B)],
                              osem[p]).wait()

    def blend(i, p):
        def do_point(b, lane):
            blk = lax.mul(lax.div(b, LANES), LANES)
            accs = [jnp.zeros((LANES,), jnp.float32)] * CH
            for k in range(4):
                wrow_lo = w_v[p, 2 * k, pl.ds(blk, LANES)]
                wrow_hi = w_v[p, 2 * k + 1, pl.ds(blk, LANES)]
                wlo = wrow_lo.at[lane].get(mode="promise_in_bounds")
                whi = wrow_hi.at[lane].get(mode="promise_in_bounds")
                for t in range(CH):
                    packed = rows_v[p, k, b, pl.ds(t * LANES, LANES)]
                    a = lax.bitcast_convert_type(packed << 16, jnp.float32)
                    bv = lax.bitcast_convert_type(
                        packed & jnp.int32(-65536), jnp.float32)
                    accs[t] = accs[t] + (wlo * a + whi * bv)
            for t in range(CH):
                out_v[p, b, pl.ds(t * LANES, LANES)] = accs[t]

        def pt_body(h, _):
            b = 4 * h
            lane = jnp.full((LANES,), lax.rem(b, LANES), jnp.int32)
            for j in range(4):
                do_point(b + j, lane + j)
            return 0

        lax.fori_loop(0, B // 4, pt_body, 0)

    def half(i, p):
        @pl.when(i + 1 < NBATCH)
        def _():
            compute_and_fire(i + 1, 1 - p)

        wait_gathers(p)

        @pl.when(i >= 2)
        def _():
            wait_out(i, p)

        blend(i, p)
        pltpu.async_copy(out_v.at[p], out_hbm.at[pl.ds(base + i * B, B)],
                         osem[p])

    def round_body(g, carry):
        half(2 * g, 0)
        half(2 * g + 1, 1)
        return carry

    compute_and_fire(0, 0)
    lax.fori_loop(0, NBATCH // 2, round_body, 0)
    wait_out(NBATCH - 2, 0)
    wait_out(NBATCH - 1, 1)


@functools.partial(jax.jit, donate_argnums=())
def kernel(x, table):
    xs = x[:, 0]
    ys = x[:, 1]
    zs = x[:, 2]
    S = TABLE * TABLE * TABLE
    tt = table.reshape(D_EMBED, S).T.astype(jnp.bfloat16)
    hi = jnp.concatenate([tt[1:], tt[-1:]], axis=0)
    pair = jnp.stack([tt, hi], axis=-1)
    tab_i32 = lax.bitcast_convert_type(pair, jnp.int32)

    run = functools.partial(
        pl.kernel,
        out_type=jax.ShapeDtypeStruct((N, D_EMBED), jnp.float32),
        mesh=plsc.VectorSubcoreMesh(core_axis_name="c", subcore_axis_name="s"),
        scratch_types=[
            pltpu.VMEM((PTS,), jnp.float32),
            pltpu.VMEM((PTS,), jnp.float32),
            pltpu.VMEM((PTS,), jnp.float32),
            pltpu.VMEM((2, 4, B), jnp.int32),
            pltpu.VMEM((2, 8, B), jnp.float32),
            pltpu.VMEM((2, 4, B, WORDS), jnp.int32),
            pltpu.VMEM((2, B, D_EMBED), jnp.float32),
            pltpu.SemaphoreType.DMA,
            pltpu.SemaphoreType.DMA,
            pltpu.SemaphoreType.DMA,
            pltpu.SemaphoreType.DMA,
        ],
    )(_body)
    return run(xs, ys, zs, tab_i32)


# speedup vs baseline: 1.1184x; 1.0499x over previous
import functools

import jax
import jax.numpy as jnp
from jax import lax
from jax.experimental import pallas as pl
from jax.experimental.pallas import tpu as pltpu
from jax.experimental.pallas import tpu_sc as plsc

D_EMBED = 128
TABLE = 32
N = 262144

NC = 2
NS = 16
NW = NC * NS
PTS = N // NW
B = 64
NBATCH = PTS // B
LANES = 16
CH = D_EMBED // LANES
WORDS = D_EMBED


def _pair_terms(v, stride):
    iv = v * (TABLE / 2.0) + (TABLE - 1.0) / 2.0
    i0 = (iv + 1.0).astype(jnp.int32) - 1
    f0 = i0.astype(jnp.float32)
    w1 = iv - f0
    w0 = 1.0 - w1
    bb = jnp.clip(i0, 0, TABLE - 2)
    zero = jnp.zeros_like(w0)
    wlow = jnp.where(i0 == bb, w0, jnp.where(i0 < 0, w1, zero))
    whigh = jnp.where(i0 == bb, w1, jnp.where(i0 > TABLE - 2, w0, zero))
    return bb * stride, wlow, whigh


def _axis_terms(v, stride):
    iv = v * (TABLE / 2.0) + (TABLE - 1.0) / 2.0
    i0 = (iv + 1.0).astype(jnp.int32) - 1
    f0 = i0.astype(jnp.float32)
    w1 = iv - f0
    w0 = 1.0 - w1
    w0 = jnp.where(i0 >= 0, w0, 0.0)
    w1 = jnp.where(i0 <= TABLE - 2, w1, 0.0)
    e0 = jnp.maximum(i0, 0) * stride
    e1 = jnp.minimum(i0 + 1, TABLE - 1) * stride
    return e0, e1, w0, w1


def _body(xs_hbm, ys_hbm, zs_hbm, tab_hbm, out_hbm,
          xs_v, ys_v, zs_v, idx_v, w_v, rows_v, out_v,
          gsem0, gsem1, osem0, osem1):
    c = lax.axis_index("c")
    s = lax.axis_index("s")
    wid = s * NC + c
    base = wid * PTS
    gsem = (gsem0, gsem1)
    osem = (osem0, osem1)

    pltpu.sync_copy(xs_hbm.at[pl.ds(base, PTS)], xs_v)
    pltpu.sync_copy(ys_hbm.at[pl.ds(base, PTS)], ys_v)
    pltpu.sync_copy(zs_hbm.at[pl.ds(base, PTS)], zs_v)

    def compute_and_fire(i, p):
        b0 = i * B
        for sl in range(B // LANES):
            src = pl.ds(b0 + sl * LANES, LANES)
            dst = pl.ds(sl * LANES, LANES)
            xv = jnp.clip(xs_v[src], -1.0, 1.0)
            yv = jnp.clip(ys_v[src], -1.0, 1.0)
            zv = jnp.clip(zs_v[src], -1.0, 1.0)
            ex, wx0, wx1 = _pair_terms(xv, 1)
            ey0, ey1, wy0, wy1 = _axis_terms(yv, TABLE)
            ez0, ez1, wz0, wz1 = _axis_terms(zv, TABLE * TABLE)
            for k in range(4):
                ey, wy = (ey1, wy1) if (k & 1) else (ey0, wy0)
                ez, wz = (ez1, wz1) if (k & 2) else (ez0, wz0)
                wyz = wy * wz
                idx_v[p, k, dst] = ez + ey + ex
                w_v[p, 2 * k, dst] = wyz * wx0
                w_v[p, 2 * k + 1, dst] = wyz * wx1
        for k in range(4):
            pltpu.async_copy(tab_hbm.at[idx_v.at[p, k]], rows_v.at[p, k],
                             gsem[p])

    def wait_gathers(p):
        for k in range(4):
            pltpu.make_async_copy(tab_hbm.at[idx_v.at[p, k]],
                                  rows_v.at[p, k], gsem[p]).wait()

    def wait_out(i, p):
        pltpu.make_async_copy(out_v.at[p], out_hbm.at[pl.ds(base + i * B, B)],
                              osem[p]).wait()

    def blend(i, p):
        def do_point(b, lane, wrows):
            accs = [jnp.zeros((LANES,), jnp.float32)] * CH
            for k in range(4):
                wlo = wrows[2 * k].at[lane].get(mode="promise_in_bounds")
                whi = wrows[2 * k + 1].at[lane].get(mode="promise_in_bounds")
                for t in range(CH):
                    packed = rows_v[p, k, b, pl.ds(t * LANES, LANES)]
                    a = lax.bitcast_convert_type(packed << 16, jnp.float32)
                    bv = lax.bitcast_convert_type(
                        packed & jnp.int32(-65536), jnp.float32)
                    accs[t] = accs[t] + (wlo * a + whi * bv)
            for t in range(CH):
                out_v[p, b, pl.ds(t * LANES, LANES)] = accs[t]

        def pt_body(h, _):
            b = 4 * h
            blk = lax.mul(lax.div(b, LANES), LANES)
            lane = jnp.full((LANES,), lax.rem(b, LANES), jnp.int32)
            wrows = [w_v[p, cc, pl.ds(blk, LANES)] for cc in range(8)]
            for j in range(4):
                do_point(b + j, lane + j, wrows)
            return 0

        lax.fori_loop(0, B // 4, pt_body, 0)

    def half(i, p):
        @pl.when(i + 1 < NBATCH)
        def _():
            compute_and_fire(i + 1, 1 - p)

        wait_gathers(p)

        @pl.when(i >= 2)
        def _():
            wait_out(i, p)

        blend(i, p)
        pltpu.async_copy(out_v.at[p], out_hbm.at[pl.ds(base + i * B, B)],
                         osem[p])

    def round_body(g, carry):
        half(2 * g, 0)
        half(2 * g + 1, 1)
        return carry

    compute_and_fire(0, 0)
    lax.fori_loop(0, NBATCH // 2, round_body, 0)
    wait_out(NBATCH - 2, 0)
    wait_out(NBATCH - 1, 1)


@functools.partial(jax.jit, donate_argnums=())
def kernel(x, table):
    xs = x[:, 0]
    ys = x[:, 1]
    zs = x[:, 2]
    S = TABLE * TABLE * TABLE
    tt = table.reshape(D_EMBED, S).T.astype(jnp.bfloat16)
    hi = jnp.concatenate([tt[1:], tt[-1:]], axis=0)
    pair = jnp.stack([tt, hi], axis=-1)
    tab_i32 = lax.bitcast_convert_type(pair, jnp.int32)

    run = functools.partial(
        pl.kernel,
        out_type=jax.ShapeDtypeStruct((N, D_EMBED), jnp.float32),
        mesh=plsc.VectorSubcoreMesh(core_axis_name="c", subcore_axis_name="s"),
        scratch_types=[
            pltpu.VMEM((PTS,), jnp.float32),
            pltpu.VMEM((PTS,), jnp.float32),
            pltpu.VMEM((PTS,), jnp.float32),
            pltpu.VMEM((2, 4, B), jnp.int32),
            pltpu.VMEM((2, 8, B), jnp.float32),
            pltpu.VMEM((2, 4, B, WORDS), jnp.int32),
            pltpu.VMEM((2, B, D_EMBED), jnp.float32),
            pltpu.SemaphoreType.DMA,
            pltpu.SemaphoreType.DMA,
            pltpu.SemaphoreType.DMA,
            pltpu.SemaphoreType.DMA,
        ],
    )(_body)
    return run(xs, ys, zs, tab_i32)
